# baseline (speedup 1.0000x reference)
import functools

import jax
import jax.numpy as jnp
from jax import lax
from jax.experimental import pallas as pl
from jax.experimental.pallas import tpu as pltpu
from jax.experimental.pallas import tpu_sc as plsc

_N, _C, _H, _W = 64, 256, 56, 56
_ROWS = _N * _H * _W
_NC, _NS, _L = 2, 16, 16
_NW = _NC * _NS
_RPW = _ROWS // _NW
_R = 112
_NT = _RPW // _R
_CSETS = _C // _L
_UNROLL = 8


def _nz_body(mask_ref, sel_ref):
    m = mask_ref[0, :] != 0.0
    row = lax.broadcasted_iota(jnp.int32, (_C, _C), 0)
    col = lax.broadcasted_iota(jnp.int32, (_C, _C), 1)
    mcol = jnp.broadcast_to(m[None, :], (_C, _C))
    rank = jnp.sum(jnp.where(mcol & (col < row), 1, 0), axis=1)
    hit = mcol & (jnp.broadcast_to(rank[None, :], (_C, _C)) == row)
    sel_ref[0, :] = jnp.sum(jnp.where(hit, col, 0), axis=1)


_tc_nonzero = pl.pallas_call(
    _nz_body,
    out_shape=jax.ShapeDtypeStruct((1, _C), jnp.int32),
)


def _sc_body(x_hbm, sel_hbm, out_hbm, sel_v, ibuf, obuf, si0, si1, so0, so1):
    wid = lax.axis_index("s") * _NC + lax.axis_index("c")
    pltpu.sync_copy(sel_hbm, sel_v)
    sel16 = [sel_v[pl.ds(k * _L, _L)] for k in range(_CSETS)]

    base = wid * _RPW
    sis = (si0, si1)
    sos = (so0, so1)

    def _fill(t, b):
        pltpu.async_copy(x_hbm.at[pl.ds(base + t * _R, _R)],
                         ibuf.at[pl.ds(b * _R, _R)], sis[b])

    def _permute(b):
        ioff = b * _R

        def _rows(i, carry):
            r0 = ioff + i * _UNROLL
            for u in range(_UNROLL):
                for k in range(_CSETS):
                    g = plsc.load_gather(ibuf, [jnp.full((_L,), r0 + u,
                                                         jnp.int32), sel16[k]])
                    obuf[r0 + u, pl.ds(k * _L, _L)] = g
            return carry

        lax.fori_loop(0, _R // _UNROLL, _rows, jnp.int32(0))

    _fill(0, 0)
    _fill(1, 1)

    def _pair(i, carry):
        for b in (0, 1):
            t = 2 * i + b
            pltpu.make_async_copy(x_hbm.at[pl.ds(0, _R)],
                                  ibuf.at[pl.ds(0, _R)], sis[b]).wait()

            @pl.when(t >= 2)
            def _():
                pltpu.make_async_copy(obuf.at[pl.ds(0, _R)],
                                      out_hbm.at[pl.ds(0, _R)], sos[b]).wait()

            _permute(b)
            pltpu.async_copy(obuf.at[pl.ds(b * _R, _R)],
                             out_hbm.at[pl.ds(base + t * _R, _R)], sos[b])

            @pl.when(t + 2 < _NT)
            def _():
                _fill(t + 2, b)
        return carry

    lax.fori_loop(0, _NT // 2, _pair, jnp.int32(0))
    for b in (0, 1):
        pltpu.make_async_copy(obuf.at[pl.ds(0, _R)],
                              out_hbm.at[pl.ds(0, _R)], sos[b]).wait()


_sc_gather = functools.partial(
    pl.kernel,
    out_type=jax.ShapeDtypeStruct((_ROWS, _C), jnp.float32),
    mesh=plsc.VectorSubcoreMesh(core_axis_name="c", subcore_axis_name="s"),
    compiler_params=pltpu.CompilerParams(use_tc_tiling_on_sc=True,
                                         needs_layout_passes=False),
    scratch_types=[
        pltpu.VMEM((_C,), jnp.int32),
        pltpu.VMEM((2 * _R, _C), jnp.float32),
        pltpu.VMEM((2 * _R, _C), jnp.float32),
        pltpu.SemaphoreType.DMA,
        pltpu.SemaphoreType.DMA,
        pltpu.SemaphoreType.DMA,
        pltpu.SemaphoreType.DMA,
    ],
)(_sc_body)


def kernel(input_tensor, indexes):
    sel = _tc_nonzero(indexes.reshape(1, _C)).reshape(_C)
    xr = input_tensor.transpose(0, 2, 3, 1).reshape(_ROWS, _C)
    outr = _sc_gather(xr, sel)
    return outr.reshape(_N, _H, _W, _C).transpose(0, 3, 1, 2)


# speedup vs baseline: 1.3538x; 1.3538x over previous
import functools

import jax
import jax.numpy as jnp
from jax import lax
from jax.experimental import pallas as pl
from jax.experimental.pallas import tpu as pltpu
from jax.experimental.pallas import tpu_sc as plsc

_N, _C, _H, _W = 64, 256, 56, 56
_ROWS = _N * _H * _W
_NC, _NS, _L = 2, 16, 16
_NW = _NC * _NS
_RPW = _ROWS // _NW
_R = 112
_NT = _RPW // _R
_CSETS = _C // _L
_UNROLL = 4


def _nz_body(mask_ref, sel_ref):
    m = mask_ref[0, :] != 0.0
    row = lax.broadcasted_iota(jnp.int32, (_C, _C), 0)
    col = lax.broadcasted_iota(jnp.int32, (_C, _C), 1)
    mcol = jnp.broadcast_to(m[None, :], (_C, _C))
    rank = jnp.sum(jnp.where(mcol & (col < row), 1, 0), axis=1)
    hit = mcol & (jnp.broadcast_to(rank[None, :], (_C, _C)) == row)
    sel_ref[0, :] = jnp.sum(jnp.where(hit, col, 0), axis=1)


_tc_nonzero = pl.pallas_call(
    _nz_body,
    out_shape=jax.ShapeDtypeStruct((1, _C), jnp.int32),
)


def _sc_body(x_hbm, sel_hbm, out_hbm, sel_v, ibuf, obuf, si0, si1, so0, so1):
    wid = lax.axis_index("s") * _NC + lax.axis_index("c")
    pltpu.sync_copy(sel_hbm, sel_v)
    sel16 = [sel_v[pl.ds(k * _L, _L)] for k in range(_CSETS)]

    base = wid * _RPW
    sis = (si0, si1)
    sos = (so0, so1)

    def _fill(t, b):
        pltpu.async_copy(x_hbm.at[pl.ds(base + t * _R, _R)],
                         ibuf.at[pl.ds(b * _R, _R)], sis[b])

    def _permute(b):
        ioff = b * _R

        def _rows(i, carry):
            r0 = ioff + i * _UNROLL
            for u in range(_UNROLL):
                for k in range(_CSETS):
                    g = plsc.load_gather(ibuf, [jnp.full((_L,), r0 + u,
                                                         jnp.int32), sel16[k]])
                    obuf[r0 + u, pl.ds(k * _L, _L)] = g
            return carry

        lax.fori_loop(0, _R // _UNROLL, _rows, jnp.int32(0))

    _fill(0, 0)
    _fill(1, 1)

    def _pair(i, carry):
        for b in (0, 1):
            t = 2 * i + b
            pltpu.make_async_copy(x_hbm.at[pl.ds(0, _R)],
                                  ibuf.at[pl.ds(0, _R)], sis[b]).wait()

            @pl.when(t >= 2)
            def _():
                pltpu.make_async_copy(obuf.at[pl.ds(0, _R)],
                                      out_hbm.at[pl.ds(0, _R)], sos[b]).wait()

            _permute(b)
            pltpu.async_copy(obuf.at[pl.ds(b * _R, _R)],
                             out_hbm.at[pl.ds(base + t * _R, _R)], sos[b])

            @pl.when(t + 2 < _NT)
            def _():
                _fill(t + 2, b)
        return carry

    lax.fori_loop(0, _NT // 2, _pair, jnp.int32(0))
    for b in (0, 1):
        pltpu.make_async_copy(obuf.at[pl.ds(0, _R)],
                              out_hbm.at[pl.ds(0, _R)], sos[b]).wait()


_sc_gather = functools.partial(
    pl.kernel,
    out_type=jax.ShapeDtypeStruct((_ROWS, _C), jnp.float32),
    mesh=plsc.VectorSubcoreMesh(core_axis_name="c", subcore_axis_name="s"),
    compiler_params=pltpu.CompilerParams(use_tc_tiling_on_sc=True,
                                         needs_layout_passes=False),
    scratch_types=[
        pltpu.VMEM((_C,), jnp.int32),
        pltpu.VMEM((2 * _R, _C), jnp.float32),
        pltpu.VMEM((2 * _R, _C), jnp.float32),
        pltpu.SemaphoreType.DMA,
        pltpu.SemaphoreType.DMA,
        pltpu.SemaphoreType.DMA,
        pltpu.SemaphoreType.DMA,
    ],
)(_sc_body)


def kernel(input_tensor, indexes):
    sel = _tc_nonzero(indexes.reshape(1, _C)).reshape(_C)
    xr = input_tensor.transpose(0, 2, 3, 1).reshape(_ROWS, _C)
    outr = _sc_gather(xr, sel)
    return outr.reshape(_N, _H, _W, _C).transpose(0, 3, 1, 2)


# speedup vs baseline: 2.8752x; 2.1239x over previous
import functools

import jax
import jax.numpy as jnp
from jax import lax
from jax.experimental import pallas as pl
from jax.experimental.pallas import tpu as pltpu
from jax.experimental.pallas import tpu_sc as plsc

_N, _C, _H, _W = 64, 256, 56, 56
_ROWS = _N * _H * _W
_NC, _NS, _L = 2, 16, 16
_NW = _NC * _NS
_RPW = _ROWS // _NW
_R = 112
_NT = _RPW // _R
_CSETS = _C // _L
_UNROLL = 4


def _nz_body(mask_ref, sel_ref):
    m = mask_ref[0, :] != 0.0
    row = lax.broadcasted_iota(jnp.int32, (_C, _C), 0)
    col = lax.broadcasted_iota(jnp.int32, (_C, _C), 1)
    mcol = jnp.broadcast_to(m[None, :], (_C, _C))
    rank = jnp.sum(jnp.where(mcol & (col < row), 1, 0), axis=1)
    hit = mcol & (jnp.broadcast_to(rank[None, :], (_C, _C)) == row)
    sel_ref[0, :] = jnp.sum(jnp.where(hit, col, 0), axis=1)


_tc_nonzero = pl.pallas_call(
    _nz_body,
    out_shape=jax.ShapeDtypeStruct((1, _C), jnp.int32),
)


def _sc_body(x_hbm, sel_hbm, out_hbm, sel_v, ibuf, obuf, si0, si1, so0, so1):
    wid = lax.axis_index("s") * _NC + lax.axis_index("c")
    pltpu.sync_copy(sel_hbm, sel_v)
    sel16 = [sel_v[pl.ds(k * _L, _L)] for k in range(_CSETS)]

    base = wid * _RPW
    sis = (si0, si1)
    sos = (so0, so1)

    def _fill(t, b):
        pltpu.async_copy(x_hbm.at[pl.ds(base + t * _R, _R)],
                         ibuf.at[pl.ds(b * _R, _R)], sis[b])

    def _permute(b):
        ioff = b * _R

        def _rows(i, carry):
            r0 = ioff + i * _UNROLL
            for u in range(_UNROLL):
                row = jnp.full((_L,), r0 + u, jnp.int32)
                g = [plsc.load_gather(ibuf, [row, sel16[k]])
                     for k in range(_CSETS)]
                for k in range(_CSETS):
                    obuf[r0 + u, pl.ds(k * _L, _L)] = g[k]
            return carry

        lax.fori_loop(0, _R // _UNROLL, _rows, jnp.int32(0))

    _fill(0, 0)
    _fill(1, 1)

    def _pair(i, carry):
        for b in (0, 1):
            t = 2 * i + b
            pltpu.make_async_copy(x_hbm.at[pl.ds(0, _R)],
                                  ibuf.at[pl.ds(0, _R)], sis[b]).wait()

            @pl.when(t >= 2)
            def _():
                pltpu.make_async_copy(obuf.at[pl.ds(0, _R)],
                                      out_hbm.at[pl.ds(0, _R)], sos[b]).wait()

            _permute(b)
            pltpu.async_copy(obuf.at[pl.ds(b * _R, _R)],
                             out_hbm.at[pl.ds(base + t * _R, _R)], sos[b])

            @pl.when(t + 2 < _NT)
            def _():
                _fill(t + 2, b)
        return carry

    lax.fori_loop(0, _NT // 2, _pair, jnp.int32(0))
    for b in (0, 1):
        pltpu.make_async_copy(obuf.at[pl.ds(0, _R)],
                              out_hbm.at[pl.ds(0, _R)], sos[b]).wait()


_sc_gather = functools.partial(
    pl.kernel,
    out_type=jax.ShapeDtypeStruct((_ROWS, _C), jnp.float32),
    mesh=plsc.VectorSubcoreMesh(core_axis_name="c", subcore_axis_name="s"),
    compiler_params=pltpu.CompilerParams(use_tc_tiling_on_sc=True,
                                         needs_layout_passes=False),
    scratch_types=[
        pltpu.VMEM((_C,), jnp.int32),
        pltpu.VMEM((2 * _R, _C), jnp.float32),
        pltpu.VMEM((2 * _R, _C), jnp.float32),
        pltpu.SemaphoreType.DMA,
        pltpu.SemaphoreType.DMA,
        pltpu.SemaphoreType.DMA,
        pltpu.SemaphoreType.DMA,
    ],
)(_sc_body)


def kernel(input_tensor, indexes):
    sel = _tc_nonzero(indexes.reshape(1, _C)).reshape(_C)
    xr = input_tensor.transpose(0, 2, 3, 1).reshape(_ROWS, _C)
    outr = _sc_gather(xr, sel)
    return outr.reshape(_N, _H, _W, _C).transpose(0, 3, 1, 2)
